# Initial kernel scaffold; baseline (speedup 1.0000x reference)
import jax
import jax.numpy as jnp
from jax.experimental import pallas as pl


def kernel(ged, y, coef_dist):
    raise NotImplementedError("write your pallas kernel here")


# speedup vs baseline: 10.2272x; 10.2272x over previous
import functools

import jax
import jax.numpy as jnp
from jax import lax
from jax.experimental import pallas as pl
from jax.experimental.pallas import tpu as pltpu
from jax.experimental.pallas import tpu_sc as plsc

NB_TEST = 1024
N_CAND = 16384
K = 64
LANES = 16
NUM_CORES = 2
NUM_SUBCORES = 16
NUM_WORKERS = NUM_CORES * NUM_SUBCORES
ROWS_PER_WORKER = NB_TEST // NUM_WORKERS
VREGS_PER_ROW = N_CAND // LANES
SCAN_UNROLL = 8
THRESH0 = 160.0 / N_CAND
MAX_FINITE_BITS = 0x7F7FFFFF


def _sc_body(ged_h, y_h, coef_h, out_h,
             rb0, rb1, cvals, cidx, y_v, coef_v, out_v, sem0, sem1):
    wid = lax.axis_index("s") * NUM_CORES + lax.axis_index("c")
    base_row = wid * ROWS_PER_WORKER

    pltpu.sync_copy(y_h, y_v)
    pltpu.sync_copy(coef_h, coef_v)
    coef = coef_v[...]
    alpha = coef * coef

    lane = lax.iota(jnp.int32, LANES)
    zf = jnp.zeros((LANES,), jnp.float32)
    zi = jnp.zeros((LANES,), jnp.int32)

    def scan_pass(rowbuf, t_vec):
        def sbody(i, off):
            for u in range(SCAN_UNROLL):
                j = i * SCAN_UNROLL + u
                v = rowbuf[pl.ds(j * LANES, LANES)]
                m = v < t_vec
                pref = plsc.cumsum(m.astype(jnp.int32))
                pos = off + pref - 1
                plsc.store_scatter(cvals, [pos], v, mask=m)
                iv = lane + j * LANES
                plsc.store_scatter(cidx, [pos], iv, mask=m)
                off = off + plsc.all_reduce_population_count(m)
            return off
        return lax.fori_loop(0, VREGS_PER_ROW // SCAN_UNROLL, sbody, zi)

    def process_row(rowbuf):
        t0_vec = jnp.full((LANES,), THRESH0, jnp.float32)
        off = scan_pass(rowbuf, t0_vec)
        cnt = jnp.max(off)

        @pl.when(cnt < K)
        def _fallback():
            scan_pass(rowbuf, jnp.full((LANES,), jnp.inf, jnp.float32))

        cnt = jnp.where(cnt < K, N_CAND, cnt)
        nv = (cnt + LANES - 1) >> 4

        def count_le(mid):
            def cbody(j, acc):
                v = cvals[pl.ds(j * LANES, LANES)]
                kb = plsc.bitcast(v, jnp.int32)
                valid = (lane + j * LANES) < cnt
                mle = valid & (kb <= mid)
                return acc + plsc.all_reduce_population_count(mle)
            return jnp.max(lax.fori_loop(0, nv, cbody, zi))

        def bbody(_, lh):
            lo, hi = lh
            mid = lo + ((hi - lo) >> 1)
            c = count_le(mid)
            ge = c >= K
            return jnp.where(ge, lo, mid + 1), jnp.where(ge, mid, hi)

        t_bits, _ = lax.fori_loop(0, 31, bbody,
                                  (jnp.int32(0), jnp.int32(MAX_FINITE_BITS)))
        n_less = count_le(t_bits - 1)
        m_need = K - n_less

        def fbody(j, carry):
            num, den, run_eq = carry
            v = cvals[pl.ds(j * LANES, LANES)]
            kb = plsc.bitcast(v, jnp.int32)
            valid = (lane + j * LANES) < cnt
            eq = valid & (kb == t_bits)
            pref = plsc.cumsum(eq.astype(jnp.int32)) + run_eq
            sel = (valid & (kb < t_bits)) | (eq & (pref <= m_need))
            iv = cidx[pl.ds(j * LANES, LANES)]
            yg = plsc.load_gather(y_v, [iv], mask=sel)
            yg = jnp.where(sel, yg, 0.0)
            s = jnp.where(sel, jnp.exp(-alpha * v), 0.0)
            run_eq = run_eq + plsc.all_reduce_population_count(eq)
            return num + s * yg, den + s, run_eq

        num, den, _ = lax.fori_loop(0, nv, fbody, (zf, zf, zi))
        num_b = jnp.broadcast_to(jnp.sum(num), (LANES,))
        den_b = jnp.broadcast_to(jnp.sum(den), (LANES,))
        return num_b / den_b

    def row_slice(r):
        return ged_h.at[pl.ds((base_row + r) * N_CAND, N_CAND)]

    pltpu.make_async_copy(row_slice(0), rb0, sem0).start()

    def step(g, carry):
        out0, out1 = carry
        r0 = 2 * g

        def put(out0, out1, r, val_vec):
            in0 = r < NUM_SUBCORES
            sel0 = (lane == r) & in0
            sel1 = (lane == (r - NUM_SUBCORES)) & jnp.logical_not(in0)
            return (jnp.where(sel0, val_vec, out0),
                    jnp.where(sel1, val_vec, out1))

        pltpu.make_async_copy(row_slice(r0 + 1), rb1, sem1).start()
        pltpu.make_async_copy(row_slice(r0), rb0, sem0).wait()
        out0, out1 = put(out0, out1, r0, process_row(rb0))

        @pl.when(g < ROWS_PER_WORKER // 2 - 1)
        def _prefetch():
            pltpu.make_async_copy(row_slice(r0 + 2), rb0, sem0).start()

        pltpu.make_async_copy(row_slice(r0 + 1), rb1, sem1).wait()
        out0, out1 = put(out0, out1, r0 + 1, process_row(rb1))
        return out0, out1

    out0, out1 = lax.fori_loop(0, ROWS_PER_WORKER // 2, step, (zf, zf))
    out_v[pl.ds(0, LANES)] = out0
    out_v[pl.ds(LANES, LANES)] = out1
    pltpu.sync_copy(out_v, out_h.at[pl.ds(base_row, ROWS_PER_WORKER)])


_sc_kernel = functools.partial(
    pl.kernel,
    mesh=plsc.VectorSubcoreMesh(core_axis_name="c", subcore_axis_name="s"),
    out_type=jax.ShapeDtypeStruct((NB_TEST,), jnp.float32),
    compiler_params=pltpu.CompilerParams(needs_layout_passes=False),
    scratch_types=[
        pltpu.VMEM((N_CAND,), jnp.float32),
        pltpu.VMEM((N_CAND,), jnp.float32),
        pltpu.VMEM((N_CAND,), jnp.float32),
        pltpu.VMEM((N_CAND,), jnp.int32),
        pltpu.VMEM((N_CAND,), jnp.float32),
        pltpu.VMEM((LANES,), jnp.float32),
        pltpu.VMEM((NUM_WORKERS,), jnp.float32),
        pltpu.SemaphoreType.DMA,
        pltpu.SemaphoreType.DMA,
    ],
)(_sc_body)


@jax.jit
def kernel(ged, y, coef_dist):
    coef16 = jnp.broadcast_to(coef_dist.astype(jnp.float32), (LANES,))
    return _sc_kernel(ged, y, coef16)


# speedup vs baseline: 29.1984x; 2.8550x over previous
import functools

import jax
import jax.numpy as jnp
from jax import lax
from jax.experimental import pallas as pl
from jax.experimental.pallas import tpu as pltpu
from jax.experimental.pallas import tpu_sc as plsc

NB_TEST = 1024
N_CAND = 16384
K = 64
LANES = 16
NUM_CORES = 2
NUM_SUBCORES = 16
NUM_WORKERS = NUM_CORES * NUM_SUBCORES
ROWS_PER_WORKER = NB_TEST // NUM_WORKERS
VREGS_PER_ROW = N_CAND // LANES
SCAN_UNROLL = 8
THRESH0 = 160.0 / N_CAND
MAX_FINITE_BITS = 0x7F7FFFFF


def _sc_body(ged_h, y_h, coef_h, out_h,
             rb0, rb1, cidx, y_v, coef_v, out_v, sem0, sem1):
    wid = lax.axis_index("s") * NUM_CORES + lax.axis_index("c")
    base_row = wid * ROWS_PER_WORKER

    pltpu.sync_copy(y_h, y_v)
    pltpu.sync_copy(coef_h, coef_v)
    coef = coef_v[...]
    alpha = coef * coef

    lane = lax.iota(jnp.int32, LANES)
    zf = jnp.zeros((LANES,), jnp.float32)
    zi = jnp.zeros((LANES,), jnp.int32)

    def scan_pass(rowbuf, t_vec):
        def sbody(i, off):
            base = i * SCAN_UNROLL
            vs = [rowbuf[pl.ds((base + u) * LANES, LANES)]
                  for u in range(SCAN_UNROLL)]
            ms = [v < t_vec for v in vs]
            cums = [plsc.cumsum(m.astype(jnp.int32)) for m in ms]
            cnts = [plsc.all_reduce_population_count(m) for m in ms]
            offs = [off]
            for u in range(SCAN_UNROLL):
                offs.append(offs[u] + cnts[u])
            for u in range(SCAN_UNROLL):
                pos = offs[u] + cums[u] - 1
                iv = lane + (base + u) * LANES
                plsc.store_scatter(cidx, [pos], iv, mask=ms[u])
            return offs[SCAN_UNROLL]
        return lax.fori_loop(0, VREGS_PER_ROW // SCAN_UNROLL, sbody, zi)

    def process_row(rowbuf):
        t0_vec = jnp.full((LANES,), THRESH0, jnp.float32)
        off = scan_pass(rowbuf, t0_vec)
        cnt = jnp.max(off)

        @pl.when(cnt < K)
        def _fallback():
            scan_pass(rowbuf, jnp.full((LANES,), jnp.inf, jnp.float32))

        cnt = jnp.where(cnt < K, N_CAND, cnt)
        nv = (cnt + LANES - 1) >> 4

        def cand_vals_at(j, valid):
            iv = cidx[pl.ds(j * LANES, LANES)]
            v = plsc.load_gather(rowbuf, [iv], mask=valid)
            return iv, plsc.bitcast(v, jnp.int32), v

        def count_le(mid_vec):
            def cbody(j, acc):
                valid = (lane + j * LANES) < cnt
                _, kb, _ = cand_vals_at(j, valid)
                mle = valid & (kb <= mid_vec)
                return acc + plsc.all_reduce_population_count(mle)
            return lax.fori_loop(0, nv, cbody, zi)

        def bbody(_, lh):
            lo, hi = lh
            mid = lo + ((hi - lo) >> 1)
            ge = count_le(mid) >= K
            return jnp.where(ge, lo, mid + 1), jnp.where(ge, mid, hi)

        t_bits, _ = lax.fori_loop(
            0, 31, bbody,
            (zi, jnp.full((LANES,), MAX_FINITE_BITS, jnp.int32)))
        n_less = count_le(t_bits - 1)
        m_need = K - n_less

        def fbody(j, carry):
            num, den, run_eq = carry
            valid = (lane + j * LANES) < cnt
            iv, kb, v = cand_vals_at(j, valid)
            eq = valid & (kb == t_bits)
            pref = plsc.cumsum(eq.astype(jnp.int32)) + run_eq
            sel = (valid & (kb < t_bits)) | (eq & (pref <= m_need))
            yg = plsc.load_gather(y_v, [iv], mask=sel)
            yg = jnp.where(sel, yg, 0.0)
            s = jnp.where(sel, jnp.exp(-alpha * v), 0.0)
            run_eq = run_eq + plsc.all_reduce_population_count(eq)
            return num + s * yg, den + s, run_eq

        num, den, _ = lax.fori_loop(0, nv, fbody, (zf, zf, zi))
        num_b = jnp.broadcast_to(jnp.sum(num), (LANES,))
        den_b = jnp.broadcast_to(jnp.sum(den), (LANES,))
        return num_b / den_b

    def row_slice(r):
        return ged_h.at[pl.ds((base_row + r) * N_CAND, N_CAND)]

    pltpu.make_async_copy(row_slice(0), rb0, sem0).start()

    def step(g, carry):
        out0, out1 = carry
        r0 = 2 * g

        def put(out0, out1, r, val_vec):
            in0 = r < NUM_SUBCORES
            sel0 = (lane == r) & in0
            sel1 = (lane == (r - NUM_SUBCORES)) & jnp.logical_not(in0)
            return (jnp.where(sel0, val_vec, out0),
                    jnp.where(sel1, val_vec, out1))

        pltpu.make_async_copy(row_slice(r0 + 1), rb1, sem1).start()
        pltpu.make_async_copy(row_slice(r0), rb0, sem0).wait()
        out0, out1 = put(out0, out1, r0, process_row(rb0))

        @pl.when(g < ROWS_PER_WORKER // 2 - 1)
        def _prefetch():
            pltpu.make_async_copy(row_slice(r0 + 2), rb0, sem0).start()

        pltpu.make_async_copy(row_slice(r0 + 1), rb1, sem1).wait()
        out0, out1 = put(out0, out1, r0 + 1, process_row(rb1))
        return out0, out1

    out0, out1 = lax.fori_loop(0, ROWS_PER_WORKER // 2, step, (zf, zf))
    out_v[pl.ds(0, LANES)] = out0
    out_v[pl.ds(LANES, LANES)] = out1
    pltpu.sync_copy(out_v, out_h.at[pl.ds(base_row, ROWS_PER_WORKER)])


_sc_kernel = functools.partial(
    pl.kernel,
    mesh=plsc.VectorSubcoreMesh(core_axis_name="c", subcore_axis_name="s"),
    out_type=jax.ShapeDtypeStruct((NB_TEST,), jnp.float32),
    compiler_params=pltpu.CompilerParams(needs_layout_passes=False),
    scratch_types=[
        pltpu.VMEM((N_CAND,), jnp.float32),
        pltpu.VMEM((N_CAND,), jnp.float32),
        pltpu.VMEM((N_CAND,), jnp.int32),
        pltpu.VMEM((N_CAND,), jnp.float32),
        pltpu.VMEM((LANES,), jnp.float32),
        pltpu.VMEM((NUM_WORKERS,), jnp.float32),
        pltpu.SemaphoreType.DMA,
        pltpu.SemaphoreType.DMA,
    ],
)(_sc_body)


@jax.jit
def kernel(ged, y, coef_dist):
    coef16 = jnp.broadcast_to(coef_dist.astype(jnp.float32), (LANES,))
    return _sc_kernel(ged, y, coef16)
